# baseline (speedup 1.0000x reference)
import jax
import jax.numpy as jnp
from jax import lax
from jax.experimental import pallas as pl
from jax.experimental.pallas import tpu as pltpu
from jax.experimental.pallas import tpu_sc as plsc

EMBED = 32
BATCH = 16384

NUM_CORES = 2
NUM_SUBCORES = 16
LANES = 16
NUM_WORKERS = NUM_CORES * NUM_SUBCORES
B_PER_W = BATCH // NUM_WORKERS
CHUNK = 128
NCHUNKS = B_PER_W // CHUNK
NBUF = 2


def _mf_body(u_hbm, i_hbm, user_hbm, item_hbm, out_hbm,
             u_idx, i_idx, u_rows, i_rows, out_v, sem_u, sem_i):
    wid = lax.axis_index("s") * NUM_CORES + lax.axis_index("c")
    base = wid * B_PER_W

    pltpu.sync_copy(u_hbm.at[pl.ds(base, B_PER_W)], u_idx)
    pltpu.sync_copy(i_hbm.at[pl.ds(base, B_PER_W)], i_idx)

    lane = lax.iota(jnp.int32, LANES)

    def issue_chunk(c, buf):
        def issue_group(g, carry):
            gvec = c * CHUNK + g * LANES + lane
            uvec = plsc.load_gather(u_idx, [gvec])
            ivec = plsc.load_gather(i_idx, [gvec])
            for l in range(LANES):
                b = g * LANES + l
                pltpu.async_copy(user_hbm.at[pl.ds(uvec[l], 1), :],
                                 u_rows.at[buf, pl.ds(b, 1), :], sem_u)
                pltpu.async_copy(item_hbm.at[pl.ds(ivec[l], 1), :],
                                 i_rows.at[buf, pl.ds(b, 1), :], sem_i)
            return carry

        lax.fori_loop(0, CHUNK // LANES, issue_group, 0)

    def drain_chunk(buf):
        pltpu.make_async_copy(user_hbm.at[pl.ds(0, CHUNK), :],
                              u_rows.at[buf], sem_u).wait()
        pltpu.make_async_copy(item_hbm.at[pl.ds(0, CHUNK), :],
                              i_rows.at[buf], sem_i).wait()

    def compute_chunk(c, buf):
        def compute_group(g, carry):
            bvec = g * LANES + lane
            acc = jnp.zeros((LANES,), jnp.float32)
            for d in range(EMBED):
                dvec = jnp.full((LANES,), d, jnp.int32)
                uv = plsc.load_gather(u_rows.at[buf], [bvec, dvec])
                iv = plsc.load_gather(i_rows.at[buf], [bvec, dvec])
                acc = acc + uv * iv
            sig = 1.0 / (1.0 + jnp.exp(-acc))
            plsc.store_scatter(out_v, [c * CHUNK + bvec], sig)
            return carry

        lax.fori_loop(0, CHUNK // LANES, compute_group, 0)

    issue_chunk(0, 0)
    for c in range(NCHUNKS):
        buf = c % NBUF
        drain_chunk(buf)
        if c + 1 < NCHUNKS:
            issue_chunk(c + 1, (c + 1) % NBUF)
        compute_chunk(c, buf)

    pltpu.sync_copy(out_v, out_hbm.at[pl.ds(base, B_PER_W)])


def kernel(u, i, user_table, item_table):
    u = u.astype(jnp.int32)
    i = i.astype(jnp.int32)
    mesh = plsc.VectorSubcoreMesh(core_axis_name="c", subcore_axis_name="s")
    k = pl.kernel(
        _mf_body,
        out_type=jax.ShapeDtypeStruct((BATCH,), jnp.float32),
        mesh=mesh,
        compiler_params=pltpu.CompilerParams(needs_layout_passes=False),
        scratch_types=[
            pltpu.VMEM((B_PER_W,), jnp.int32),
            pltpu.VMEM((B_PER_W,), jnp.int32),
            pltpu.VMEM((NBUF, CHUNK, EMBED), jnp.float32),
            pltpu.VMEM((NBUF, CHUNK, EMBED), jnp.float32),
            pltpu.VMEM((B_PER_W,), jnp.float32),
            pltpu.SemaphoreType.DMA,
            pltpu.SemaphoreType.DMA,
        ],
    )
    return k(u, i, user_table, item_table)


# speedup vs baseline: 1.0012x; 1.0012x over previous
import jax
import jax.numpy as jnp
from jax import lax
from jax.experimental import pallas as pl
from jax.experimental.pallas import tpu as pltpu
from jax.experimental.pallas import tpu_sc as plsc

EMBED = 32
BATCH = 16384

NUM_CORES = 2
NUM_SUBCORES = 16
LANES = 16
NUM_WORKERS = NUM_CORES * NUM_SUBCORES
B_PER_W = BATCH // NUM_WORKERS
CHUNK = 128
NCHUNKS = B_PER_W // CHUNK
NBUF = 2


def _mf_body(u_hbm, i_hbm, user_hbm, item_hbm, out_hbm,
             u_idx, i_idx, u_rows, i_rows, out_v, sem_u, sem_i):
    wid = lax.axis_index("s") * NUM_CORES + lax.axis_index("c")
    base = wid * B_PER_W

    pltpu.sync_copy(u_hbm.at[pl.ds(base, B_PER_W)], u_idx)
    pltpu.sync_copy(i_hbm.at[pl.ds(base, B_PER_W)], i_idx)

    lane = lax.iota(jnp.int32, LANES)

    def issue_chunk(c, buf):
        def issue_group(g, carry):
            gvec = c * CHUNK + g * LANES + lane
            uvec = plsc.load_gather(u_idx, [gvec])
            ivec = plsc.load_gather(i_idx, [gvec])
            for l in range(LANES):
                b = g * LANES + l
                pltpu.async_copy(user_hbm.at[pl.ds(uvec[l], 1), :],
                                 u_rows.at[buf, pl.ds(b, 1), :], sem_u)
                pltpu.async_copy(item_hbm.at[pl.ds(ivec[l], 1), :],
                                 i_rows.at[buf, pl.ds(b, 1), :], sem_i)
            return carry

        lax.fori_loop(0, CHUNK // LANES, issue_group, 0)

    def drain_chunk(buf):
        pltpu.make_async_copy(user_hbm.at[pl.ds(0, CHUNK), :],
                              u_rows.at[buf], sem_u).wait()
        pltpu.make_async_copy(item_hbm.at[pl.ds(0, CHUNK), :],
                              i_rows.at[buf], sem_i).wait()

    def compute_chunk(c, buf):
        def compute_group(g, carry):
            bvec = g * LANES + lane
            acc = jnp.zeros((LANES,), jnp.float32)
            for d in range(EMBED):
                dvec = jnp.full((LANES,), d, jnp.int32)
                uv = plsc.load_gather(u_rows.at[buf], [bvec, dvec])
                iv = plsc.load_gather(i_rows.at[buf], [bvec, dvec])
                acc = acc + uv * iv
            sig = 1.0 / (1.0 + jnp.exp(-acc))
            plsc.store_scatter(out_v, [c * CHUNK + bvec], sig)
            return carry

        lax.fori_loop(0, CHUNK // LANES, compute_group, 0)

    with jax.named_scope("issue0"):
        issue_chunk(0, 0)
    for c in range(NCHUNKS):
        buf = c % NBUF
        with jax.named_scope("drain"):
            drain_chunk(buf)
        if c + 1 < NCHUNKS:
            with jax.named_scope("issue"):
                issue_chunk(c + 1, (c + 1) % NBUF)
        with jax.named_scope("compute"):
            compute_chunk(c, buf)

    pltpu.sync_copy(out_v, out_hbm.at[pl.ds(base, B_PER_W)])


def kernel(u, i, user_table, item_table):
    u = u.astype(jnp.int32)
    i = i.astype(jnp.int32)
    mesh = plsc.VectorSubcoreMesh(core_axis_name="c", subcore_axis_name="s")
    k = pl.kernel(
        _mf_body,
        out_type=jax.ShapeDtypeStruct((BATCH,), jnp.float32),
        mesh=mesh,
        compiler_params=pltpu.CompilerParams(needs_layout_passes=False),
        scratch_types=[
            pltpu.VMEM((B_PER_W,), jnp.int32),
            pltpu.VMEM((B_PER_W,), jnp.int32),
            pltpu.VMEM((NBUF, CHUNK, EMBED), jnp.float32),
            pltpu.VMEM((NBUF, CHUNK, EMBED), jnp.float32),
            pltpu.VMEM((B_PER_W,), jnp.float32),
            pltpu.SemaphoreType.DMA,
            pltpu.SemaphoreType.DMA,
        ],
    )
    return k(u, i, user_table, item_table)


# speedup vs baseline: 1.0198x; 1.0186x over previous
import jax
import jax.numpy as jnp
from jax import lax
from jax.experimental import pallas as pl
from jax.experimental.pallas import tpu as pltpu
from jax.experimental.pallas import tpu_sc as plsc

EMBED = 32
BATCH = 16384

NUM_CORES = 2
NUM_SUBCORES = 16
LANES = 16
NUM_WORKERS = NUM_CORES * NUM_SUBCORES
B_PER_W = BATCH // NUM_WORKERS
CHUNK = 128
NCHUNKS = B_PER_W // CHUNK
NBUF = 2


def _mf_body(u_hbm, i_hbm, user_hbm, item_hbm, out_hbm,
             u_idx, i_idx, u_rows, i_rows, out_v, sem_u, sem_i):
    wid = lax.axis_index("s") * NUM_CORES + lax.axis_index("c")
    base = wid * B_PER_W

    pltpu.sync_copy(u_hbm.at[pl.ds(base, B_PER_W)], u_idx)
    pltpu.sync_copy(i_hbm.at[pl.ds(base, B_PER_W)], i_idx)

    def issue_chunk(c, buf):
        def issue_group(g, carry):
            uvec = u_idx[pl.ds(c * CHUNK + g * LANES, LANES)]
            ivec = i_idx[pl.ds(c * CHUNK + g * LANES, LANES)]
            for l in range(LANES):
                ru = uvec[l]
                ri = ivec[l]
                b = g * LANES + l
                pltpu.async_copy(user_hbm.at[pl.ds(ru, 1), :],
                                 u_rows.at[buf, pl.ds(b, 1), :], sem_u)
                pltpu.async_copy(item_hbm.at[pl.ds(ri, 1), :],
                                 i_rows.at[buf, pl.ds(b, 1), :], sem_i)
            return carry

        lax.fori_loop(0, CHUNK // LANES, issue_group, 0)

    def drain_chunk(buf):
        pltpu.make_async_copy(user_hbm.at[pl.ds(0, CHUNK), :],
                              u_rows.at[buf], sem_u).wait()
        pltpu.make_async_copy(item_hbm.at[pl.ds(0, CHUNK), :],
                              i_rows.at[buf], sem_i).wait()

    last_lane = lax.iota(jnp.int32, LANES) == (LANES - 1)

    def compute_chunk(c, buf):
        def dot_row(b, carry):
            u0 = u_rows[buf, b, pl.ds(0, LANES)]
            u1 = u_rows[buf, b, pl.ds(LANES, LANES)]
            v0 = i_rows[buf, b, pl.ds(0, LANES)]
            v1 = i_rows[buf, b, pl.ds(LANES, LANES)]
            p = u0 * v0 + u1 * v1
            cum = plsc.cumsum(p)
            plsc.store_compressed(out_v.at[pl.ds(c * CHUNK + b, LANES)],
                                  cum, mask=last_lane)
            return carry

        lax.fori_loop(0, CHUNK, dot_row, 0, unroll=4)

    issue_chunk(0, 0)
    for c in range(NCHUNKS):
        buf = c % NBUF
        drain_chunk(buf)
        if c + 1 < NCHUNKS:
            issue_chunk(c + 1, (c + 1) % NBUF)
        compute_chunk(c, buf)

    def sigmoid_chunk(c, carry):
        s = out_v[pl.ds(c * LANES, LANES)]
        out_v[pl.ds(c * LANES, LANES)] = 1.0 / (1.0 + jnp.exp(-s))
        return carry

    lax.fori_loop(0, B_PER_W // LANES, sigmoid_chunk, 0, unroll=4)

    pltpu.sync_copy(out_v.at[pl.ds(0, B_PER_W)],
                    out_hbm.at[pl.ds(base, B_PER_W)])


def kernel(u, i, user_table, item_table):
    u = u.astype(jnp.int32)
    i = i.astype(jnp.int32)
    mesh = plsc.VectorSubcoreMesh(core_axis_name="c", subcore_axis_name="s")
    k = pl.kernel(
        _mf_body,
        out_type=jax.ShapeDtypeStruct((BATCH,), jnp.float32),
        mesh=mesh,
        compiler_params=pltpu.CompilerParams(needs_layout_passes=False),
        scratch_types=[
            pltpu.VMEM((B_PER_W,), jnp.int32),
            pltpu.VMEM((B_PER_W,), jnp.int32),
            pltpu.VMEM((NBUF, CHUNK, EMBED), jnp.float32),
            pltpu.VMEM((NBUF, CHUNK, EMBED), jnp.float32),
            pltpu.VMEM((B_PER_W + LANES,), jnp.float32),
            pltpu.SemaphoreType.DMA,
            pltpu.SemaphoreType.DMA,
        ],
    )
    return k(u, i, user_table, item_table)
